# Initial kernel scaffold; baseline (speedup 1.0000x reference)
import jax
import jax.numpy as jnp
from jax.experimental import pallas as pl


def kernel(x, mask):
    raise NotImplementedError("write your pallas kernel here")


# speedup vs baseline: 1.3409x; 1.3409x over previous
import jax
import jax.numpy as jnp
from jax.experimental import pallas as pl

PATCH = 196
EMBED = 768
B_BLK = 4


def _mask_kernel(mask_ref, x_ref, o_ref):
    m = mask_ref[...]
    idx = jax.lax.broadcasted_iota(jnp.int32, (PATCH, 1), 0)
    keep = jnp.logical_and(m == 0, idx != 0).astype(jnp.float32)
    o_ref[...] = x_ref[...] * keep[None, :, :]


def kernel(x, mask):
    batch = x.shape[0]
    out = pl.pallas_call(
        _mask_kernel,
        grid=(batch // B_BLK,),
        in_specs=[
            pl.BlockSpec((PATCH, 1), lambda i: (0, 0)),
            pl.BlockSpec((B_BLK, PATCH, EMBED), lambda i: (i, 0, 0)),
        ],
        out_specs=pl.BlockSpec((B_BLK, PATCH, EMBED), lambda i: (i, 0, 0)),
        out_shape=jax.ShapeDtypeStruct(x.shape, x.dtype),
    )(mask, x)
    return (out, mask)


# speedup vs baseline: 1.3695x; 1.0213x over previous
import jax
import jax.numpy as jnp
from jax.experimental import pallas as pl

PATCH = 196
EMBED = 768
B_BLK = 16


def _mask_kernel(mask_ref, x_ref, o_ref):
    m = mask_ref[...]
    idx = jax.lax.broadcasted_iota(jnp.int32, (PATCH, 1), 0)
    keep = jnp.logical_and(m == 0, idx != 0).astype(jnp.float32)
    o_ref[...] = x_ref[...] * keep[None, :, :]


def kernel(x, mask):
    batch = x.shape[0]
    out = pl.pallas_call(
        _mask_kernel,
        grid=(batch // B_BLK,),
        in_specs=[
            pl.BlockSpec((PATCH, 1), lambda i: (0, 0)),
            pl.BlockSpec((B_BLK, PATCH, EMBED), lambda i: (i, 0, 0)),
        ],
        out_specs=pl.BlockSpec((B_BLK, PATCH, EMBED), lambda i: (i, 0, 0)),
        out_shape=jax.ShapeDtypeStruct(x.shape, x.dtype),
    )(mask, x)
    return (out, mask)
